# baseline (speedup 1.0000x reference)
import jax
import jax.numpy as jnp
from jax.experimental import pallas as pl
from jax.experimental.pallas import tpu as pltpu

_L = 128
_SUB = 2
_B = _L * _SUB


def _lenet_kernel(c1w_ref, c1b_ref, c2w_ref, c2b_ref,
                  x_ref, w3_ref, b3_ref, f1w_ref, f1b_ref, f2w_ref, f2b_ref,
                  out_ref, p1_ref, p2_ref, rb1_ref, rb2_ref, xs_ref, p1s_ref):
    B = _L
    for sub in range(_SUB):
        lo = sub * _L
        x = x_ref[:, :, lo:lo + _L]

        for j in range(5):
            xs_ref[j] = x[:, j:j + 28, :]
        for c in range(6):
            for s in range(7):
                y0 = 4 * s
                acc = None
                for j in range(5):
                    blk = xs_ref[j, y0:y0 + 8]
                    for i in range(5):
                        w = c1w_ref[c * 25 + i * 5 + j]
                        t = w * blk[i:i + 4]
                        acc = t if acc is None else acc + t
                acc = (acc + c1b_ref[c]).reshape(2, 2, 28, B)
                rb1_ref[2 * s:2 * s + 2] = jnp.maximum(acc[:, 0], acc[:, 1])
            p = jnp.maximum(rb1_ref[:, pl.ds(0, 14, 2), :],
                            rb1_ref[:, pl.ds(1, 14, 2), :])
            p1_ref[c] = jnp.maximum(p, 0.0)

        for ci in range(6):
            p1c = p1_ref[ci]
            for j in range(5):
                p1s_ref[ci, j] = p1c[:, j:j + 10, :]
        for co in range(16):
            acc = None
            for ci in range(6):
                base = co * 150 + ci * 25
                for j in range(5):
                    blk = p1s_ref[ci, j]
                    for i in range(5):
                        w = c2w_ref[base + i * 5 + j]
                        t = w * blk[i:i + 10]
                        acc = t if acc is None else acc + t
            acc = (acc + c2b_ref[co]).reshape(5, 2, 10, B)
            rb2_ref[...] = jnp.maximum(acc[:, 0], acc[:, 1])
            p = jnp.maximum(rb2_ref[:, pl.ds(0, 5, 2), :],
                            rb2_ref[:, pl.ds(1, 5, 2), :])
            p = jnp.maximum(p, 0.0)
            for ph in range(5):
                r = co * 25 + ph * 5
                p2_ref[r:r + 5, :] = p[ph]

        f = jnp.dot(w3_ref[...], p2_ref[...],
                    preferred_element_type=jnp.float32) + b3_ref[...]
        f = jnp.maximum(f, 0.0)
        h = jnp.dot(f1w_ref[...], f,
                    preferred_element_type=jnp.float32) + f1b_ref[...]
        h = jnp.maximum(h, 0.0)
        out_ref[:, lo:lo + _L] = jnp.dot(
            f2w_ref[...], h, preferred_element_type=jnp.float32) + f2b_ref[...]


def kernel(conv1_w, conv1_b, conv2_w, conv2_b, conv3_w, conv3_b,
           fc1_w, fc1_b, fc2_w, fc2_b, x):
    n = x.shape[0]
    npad = -(-n // _B) * _B

    xt = jnp.transpose(x.astype(jnp.float32).reshape(n, 28, 28), (1, 2, 0))
    xt = jnp.pad(xt, ((2, 2), (2, 2), (0, npad - n)))

    c1w = conv1_w.astype(jnp.float32).reshape(150)
    c2w = conv2_w.astype(jnp.float32).reshape(2400)
    w3 = conv3_w.astype(jnp.float32).reshape(120, 400)
    f1w = fc1_w.astype(jnp.float32)
    f2w = fc2_w.astype(jnp.float32)

    smem = pl.BlockSpec(memory_space=pltpu.SMEM)
    full = pl.BlockSpec(memory_space=pltpu.VMEM)

    out = pl.pallas_call(
        _lenet_kernel,
        out_shape=jax.ShapeDtypeStruct((10, npad), jnp.float32),
        grid=(npad // _B,),
        in_specs=[
            smem, smem, smem, smem,
            pl.BlockSpec((32, 32, _B), lambda b: (0, 0, b)),
            full, full, full, full, full, full,
        ],
        out_specs=pl.BlockSpec((10, _B), lambda b: (0, b)),
        scratch_shapes=[
            pltpu.VMEM((6, 14, 14, _L), jnp.float32),
            pltpu.VMEM((400, _L), jnp.float32),
            pltpu.VMEM((14, 28, _L), jnp.float32),
            pltpu.VMEM((5, 10, _L), jnp.float32),
            pltpu.VMEM((5, 32, 28, _L), jnp.float32),
            pltpu.VMEM((6, 5, 14, 10, _L), jnp.float32),
        ],
        compiler_params=pltpu.CompilerParams(dimension_semantics=("parallel",)),
    )(c1w, conv1_b.astype(jnp.float32), c2w, conv2_b.astype(jnp.float32),
      xt, w3, conv3_b.astype(jnp.float32).reshape(120, 1),
      f1w, fc1_b.astype(jnp.float32).reshape(84, 1),
      f2w, fc2_b.astype(jnp.float32).reshape(10, 1))
    return out[:, :n].T


# speedup vs baseline: 1.2734x; 1.2734x over previous
import jax
import jax.numpy as jnp
from jax.experimental import pallas as pl
from jax.experimental.pallas import tpu as pltpu

_L = 128
_SUB = 1
_B = _L * _SUB


def _lenet_kernel(c1w_ref, c1b_ref, c2w_ref, c2b_ref,
                  x_ref, w3_ref, b3_ref, f1w_ref, f1b_ref, f2w_ref, f2b_ref,
                  out_ref, p1_ref, p2_ref, rb1_ref, rb2_ref, xs_ref, p1s_ref):
    B = _L
    for sub in range(_SUB):
        lo = sub * _L
        x = x_ref[:, :, lo:lo + _L]

        for j in range(5):
            xs_ref[j] = x[:, j:j + 28, :]
        for c in range(6):
            for s in range(7):
                y0 = 4 * s
                acc = None
                for j in range(5):
                    blk = xs_ref[j, y0:y0 + 8]
                    for i in range(5):
                        w = c1w_ref[c * 25 + i * 5 + j]
                        t = w * blk[i:i + 4]
                        acc = t if acc is None else acc + t
                acc = (acc + c1b_ref[c]).reshape(2, 2, 28, B)
                rb1_ref[2 * s:2 * s + 2] = jnp.maximum(acc[:, 0], acc[:, 1])
            p = jnp.maximum(rb1_ref[:, pl.ds(0, 14, 2), :],
                            rb1_ref[:, pl.ds(1, 14, 2), :])
            p1_ref[c] = jnp.maximum(p, 0.0)

        for ci in range(6):
            p1c = p1_ref[ci]
            for j in range(5):
                p1s_ref[ci, j] = p1c[:, j:j + 10, :]
        for co in range(16):
            acc = None
            for ci in range(6):
                base = co * 150 + ci * 25
                for j in range(5):
                    blk = p1s_ref[ci, j]
                    for i in range(5):
                        w = c2w_ref[base + i * 5 + j]
                        t = w * blk[i:i + 10]
                        acc = t if acc is None else acc + t
            acc = (acc + c2b_ref[co]).reshape(5, 2, 10, B)
            rb2_ref[...] = jnp.maximum(acc[:, 0], acc[:, 1])
            p = jnp.maximum(rb2_ref[:, pl.ds(0, 5, 2), :],
                            rb2_ref[:, pl.ds(1, 5, 2), :])
            p = jnp.maximum(p, 0.0)
            for ph in range(5):
                r = co * 25 + ph * 5
                p2_ref[r:r + 5, :] = p[ph]

        f = jnp.dot(w3_ref[...], p2_ref[...],
                    preferred_element_type=jnp.float32) + b3_ref[...]
        f = jnp.maximum(f, 0.0)
        h = jnp.dot(f1w_ref[...], f,
                    preferred_element_type=jnp.float32) + f1b_ref[...]
        h = jnp.maximum(h, 0.0)
        out_ref[:, lo:lo + _L] = jnp.dot(
            f2w_ref[...], h, preferred_element_type=jnp.float32) + f2b_ref[...]


def kernel(conv1_w, conv1_b, conv2_w, conv2_b, conv3_w, conv3_b,
           fc1_w, fc1_b, fc2_w, fc2_b, x):
    n = x.shape[0]
    npad = -(-n // _B) * _B

    xt = jnp.transpose(x.astype(jnp.float32).reshape(n, 28, 28), (1, 2, 0))
    xt = jnp.pad(xt, ((2, 2), (2, 2), (0, npad - n)))

    c1w = conv1_w.astype(jnp.float32).reshape(150)
    c2w = conv2_w.astype(jnp.float32).reshape(2400)
    w3 = conv3_w.astype(jnp.float32).reshape(120, 400)
    f1w = fc1_w.astype(jnp.float32)
    f2w = fc2_w.astype(jnp.float32)

    smem = pl.BlockSpec(memory_space=pltpu.SMEM)
    full = pl.BlockSpec(memory_space=pltpu.VMEM)

    out = pl.pallas_call(
        _lenet_kernel,
        out_shape=jax.ShapeDtypeStruct((10, npad), jnp.float32),
        grid=(npad // _B,),
        in_specs=[
            smem, smem, smem, smem,
            pl.BlockSpec((32, 32, _B), lambda b: (0, 0, b)),
            full, full, full, full, full, full,
        ],
        out_specs=pl.BlockSpec((10, _B), lambda b: (0, b)),
        scratch_shapes=[
            pltpu.VMEM((6, 14, 14, _L), jnp.float32),
            pltpu.VMEM((400, _L), jnp.float32),
            pltpu.VMEM((14, 28, _L), jnp.float32),
            pltpu.VMEM((5, 10, _L), jnp.float32),
            pltpu.VMEM((5, 32, 28, _L), jnp.float32),
            pltpu.VMEM((6, 5, 14, 10, _L), jnp.float32),
        ],
        compiler_params=pltpu.CompilerParams(dimension_semantics=("parallel",)),
    )(c1w, conv1_b.astype(jnp.float32), c2w, conv2_b.astype(jnp.float32),
      xt, w3, conv3_b.astype(jnp.float32).reshape(120, 1),
      f1w, fc1_b.astype(jnp.float32).reshape(84, 1),
      f2w, fc2_b.astype(jnp.float32).reshape(10, 1))
    return out[:, :n].T
